# baseline (speedup 1.0000x reference)
import functools

import jax
import jax.numpy as jnp
import numpy as np
from jax import lax
from jax.experimental import pallas as pl
from jax.experimental.pallas import tpu as pltpu
from jax.experimental.pallas import tpu_sc as plsc

F = 16
G = 3125
VOCAB = 1000
E_DIM = 64
H = 128
HEADS = 4
N = F * G
E = 93750

NPAD = 50176
CH = 6272
EPAD = 94208
EPT = EPAD // 16
GR = EPT // 128
PAD_DST = 1 << 20


def _mm_kernel(x_ref, w_ref, o_ref):
    o_ref[...] = jnp.dot(x_ref[...], w_ref[...], preferred_element_type=jnp.float32)


def _mm(x, w, block_rows=2000):
    m, k = x.shape
    _, n = w.shape
    return pl.pallas_call(
        _mm_kernel,
        grid=(m // block_rows,),
        in_specs=[
            pl.BlockSpec((block_rows, k), lambda i: (i, 0)),
            pl.BlockSpec((k, n), lambda i: (0, 0)),
        ],
        out_specs=pl.BlockSpec((block_rows, n), lambda i: (i, 0)),
        out_shape=jax.ShapeDtypeStruct((m, n), jnp.float32),
    )(x, w)


def _mmp_kernel(x_ref, w_ref, o_ref):
    o_ref[...] = jnp.dot(x_ref[...], w_ref[...],
                         preferred_element_type=jnp.float32)[None]


def _mm_planar(x_pad, w, heads):
    k = x_pad.shape[1]
    br = 6272
    return pl.pallas_call(
        _mmp_kernel,
        grid=(heads, NPAD // br),
        in_specs=[
            pl.BlockSpec((br, k), lambda h, i: (i, 0)),
            pl.BlockSpec((k, H), lambda h, i: (0, h)),
        ],
        out_specs=pl.BlockSpec((1, br, H), lambda h, i: (h, i, 0)),
        out_shape=jax.ShapeDtypeStruct((heads, NPAD, H), jnp.float32),
    )(x_pad, w)


def _cls_kernel(p_ref, w1_ref, b1_ref, w2_ref, b2_ref, o_ref):
    h1 = jnp.dot(p_ref[...], w1_ref[...], preferred_element_type=jnp.float32)
    h1 = h1 + b1_ref[...]
    h1 = jnp.where(h1 > 0, h1, 0.01 * h1)
    o_ref[...] = jnp.dot(h1, w2_ref[...], preferred_element_type=jnp.float32) + b2_ref[...]


def _make_edge_kernel(heads):
    HN = heads * NPAD
    jobs_per_sc = heads * 4

    mesh = plsc.VectorSubcoreMesh(core_axis_name="c", subcore_axis_name="s",
                                  num_cores=2, num_subcores=16)

    @functools.partial(
        pl.kernel,
        out_type=(jax.ShapeDtypeStruct((HN, H), jnp.float32),
                  jax.ShapeDtypeStruct((HN,), jnp.float32)),
        mesh=mesh,
        scratch_types=[
            pltpu.VMEM((EPT,), jnp.int32),
            pltpu.VMEM((EPT,), jnp.int32),
            pltpu.VMEM((128,), jnp.int32),
            pltpu.VMEM((128,), jnp.int32),
            pltpu.VMEM((128,), jnp.int32),
            pltpu.VMEM((128,), jnp.float32),
            pltpu.VMEM((128,), jnp.float32),
            pltpu.VMEM((128,), jnp.float32),
            pltpu.VMEM((128, H), jnp.float32),
            pltpu.VMEM((64, H), jnp.float32),
            pltpu.VMEM((128,), jnp.float32),
            pltpu.VMEM_SHARED((CH + 8, H), jnp.float32),
            pltpu.VMEM_SHARED((CH + 8,), jnp.float32),
            pltpu.SemaphoreType.DMA,
        ],
    )
    def edge_kernel(h_hbm, asrc_hbm, adst_hbm, src_hbm, dst_hbm,
                    out_hbm, den_hbm,
                    src_v, dst_v, gsrc_v, gdl_v, gdg_v, gex_v, gas_v, gad_v,
                    rows_v, zer_v, zvec_v, acc, dacc, sem):
        sc = lax.axis_index("c")
        s = lax.axis_index("s")
        zl = jnp.zeros((16,), jnp.float32)

        def zb(i, _):
            for j in range(H // 16):
                zer_v[i, pl.ds(j * 16, 16)] = zl
            return 0
        lax.fori_loop(0, 64, zb, 0)
        for j in range(8):
            zvec_v[pl.ds(j * 16, 16)] = zl

        pltpu.sync_copy(src_hbm.at[pl.ds(s * EPT, EPT)], src_v)
        pltpu.sync_copy(dst_hbm.at[pl.ds(s * EPT, EPT)], dst_v)

        def job_body(jp, _):
            job = jp * 2 + sc
            h = job >> 3 if heads == 4 else jnp.int32(0)
            c = job & 7
            lo = c * CH
            hNP = h * NPAD

            for k in range(6):
                pltpu.sync_copy(zer_v, acc.at[pl.ds(s * 392 + k * 64, 64)])
            pltpu.sync_copy(zer_v.at[pl.ds(0, 8)],
                            acc.at[pl.ds(s * 392 + 384, 8)])
            for k in range(3):
                pltpu.sync_copy(zvec_v, dacc.at[pl.ds(s * 392 + k * 128, 128)])
            pltpu.sync_copy(zvec_v.at[pl.ds(0, 8)],
                            dacc.at[pl.ds(s * 392 + 384, 8)])
            plsc.subcore_barrier()

            def gbody(g, _):
                base = g * 128
                for j in range(8):
                    sv = src_v[pl.ds(base + j * 16, 16)]
                    dv = dst_v[pl.ds(base + j * 16, 16)]
                    m = (dv >= lo) & (dv < lo + CH)
                    gsrc_v[pl.ds(j * 16, 16)] = sv + hNP
                    gdl_v[pl.ds(j * 16, 16)] = jnp.where(m, dv - lo, CH)
                    gdg_v[pl.ds(j * 16, 16)] = jnp.where(m, dv, 0) + hNP
                pltpu.async_copy(asrc_hbm.at[gsrc_v], gas_v, sem).wait()
                pltpu.async_copy(adst_hbm.at[gdg_v], gad_v, sem).wait()
                pltpu.async_copy(h_hbm.at[gsrc_v], rows_v, sem).wait()
                for j in range(8):
                    dv = dst_v[pl.ds(base + j * 16, 16)]
                    m = (dv >= lo) & (dv < lo + CH)
                    al = gas_v[pl.ds(j * 16, 16)] + gad_v[pl.ds(j * 16, 16)]
                    al = jnp.where(al > 0, al, 0.2 * al)
                    gex_v[pl.ds(j * 16, 16)] = jnp.where(m, jnp.exp(al), 0.0)

                def mb(t, _):
                    exv = gex_v[pl.ds(t * 16, 16)]
                    for l in range(16):
                        exs = jnp.full((16,), exv[l], jnp.float32)
                        for j2 in range(8):
                            r = rows_v[t * 16 + l, pl.ds(j2 * 16, 16)]
                            rows_v[t * 16 + l, pl.ds(j2 * 16, 16)] = r * exs
                    return 0
                lax.fori_loop(0, 8, mb, 0)
                pltpu.sync_copy(rows_v, acc.at[gdl_v], add=True)
                pltpu.sync_copy(gex_v, dacc.at[gdl_v], add=True)
                return 0
            lax.fori_loop(0, GR, gbody, 0)
            plsc.subcore_barrier()

            rowbase = hNP + lo + s * 392
            for k in range(3):
                pltpu.sync_copy(acc.at[pl.ds(s * 392 + k * 128, 128)],
                                out_hbm.at[pl.ds(rowbase + k * 128, 128)])
            pltpu.sync_copy(acc.at[pl.ds(s * 392 + 384, 8)],
                            out_hbm.at[pl.ds(rowbase + 384, 8)])

            @pl.when(s == 0)
            def _():
                pltpu.sync_copy(dacc.at[pl.ds(0, CH)],
                                den_hbm.at[pl.ds(hNP + lo, CH)])
            plsc.subcore_barrier()
            return 0

        lax.fori_loop(0, jobs_per_sc, job_body, 0)

    return edge_kernel


def _gat_layer(x, src_p, dst_p, W, att_src, att_dst, heads):
    x_pad = jnp.pad(x, ((0, NPAD - N), (0, 0)))
    hp = _mm_planar(x_pad, W, heads)
    Wr = W.reshape(W.shape[0], heads, H)
    ws = jnp.einsum("khc,hc->kh", Wr, att_src)
    wd = jnp.einsum("khc,hc->kh", Wr, att_dst)
    a_src = x @ ws
    a_dst = x @ wd
    asrc_p = jnp.pad(a_src.T, ((0, 0), (0, NPAD - N))).reshape(-1)
    adst_p = jnp.pad(a_dst.T, ((0, 0), (0, NPAD - N))).reshape(-1)

    ek = _make_edge_kernel(heads)
    out_sc, den_sc = ek(hp.reshape(heads * NPAD, H), asrc_p, adst_p,
                        src_p, dst_p)
    out_sc = out_sc.reshape(heads, NPAD, H)[:, :N]
    den_sc = den_sc.reshape(heads, NPAD)[:, :N]

    exl = jnp.exp(jax.nn.leaky_relu(a_src + a_dst, 0.2))
    exl_t = exl.T
    num = out_sc + hp[:, :N] * exl_t[:, :, None]
    den = den_sc + exl_t + 1e-16
    return num / den[:, :, None]


def kernel(field_indices, edge_index, batch_idx, emb_tables, aligner_W, aligner_b, mask_logits, gat1_W, gat1_att_src, gat1_att_dst, gat1_b, gat2_W, gat2_att_src, gat2_att_dst, gat2_b, cls_W1, cls_b1, cls_W2, cls_b2):
    flat_tab = emb_tables.reshape(F * VOCAB, E_DIM)
    idx = field_indices.T + (jnp.arange(F) * VOCAB)[None, :]
    emb = jnp.take(flat_tab, idx.reshape(-1), axis=0)
    gate = jax.nn.sigmoid(mask_logits)
    x = _mm(emb, aligner_W) + aligner_b
    x = x * jnp.tile(gate, G)[:, None]

    src_p = jnp.concatenate(
        [edge_index[0].astype(jnp.int32), jnp.zeros((EPAD - E,), jnp.int32)])
    dst_p = jnp.concatenate(
        [edge_index[1].astype(jnp.int32),
         jnp.full((EPAD - E,), PAD_DST, jnp.int32)])

    o1 = _gat_layer(x, src_p, dst_p, gat1_W, gat1_att_src, gat1_att_dst, HEADS)
    x1 = jnp.concatenate([o1[h] for h in range(HEADS)], axis=1) + gat1_b
    x1 = jnp.where(x1 > 0, x1, jnp.expm1(x1))

    o2 = _gat_layer(x1, src_p, dst_p, gat2_W, gat2_att_src, gat2_att_dst, 1)
    x2 = o2[0] + gat2_b

    counts = jax.ops.segment_sum(jnp.ones((N,), jnp.float32), batch_idx, num_segments=G)
    pooled = jax.ops.segment_sum(x2, batch_idx, num_segments=G)
    pooled = pooled / jnp.clip(counts, 1.0, None)[:, None]

    pooled_pad = jnp.pad(pooled, ((0, 3), (0, 0)))
    logits = pl.pallas_call(
        _cls_kernel,
        in_specs=[
            pl.BlockSpec((3128, H), lambda: (0, 0)),
            pl.BlockSpec((H, H // 2), lambda: (0, 0)),
            pl.BlockSpec((H // 2,), lambda: (0,)),
            pl.BlockSpec((H // 2, 8), lambda: (0, 0)),
            pl.BlockSpec((8,), lambda: (0,)),
        ],
        out_specs=pl.BlockSpec((3128, 8), lambda: (0, 0)),
        out_shape=jax.ShapeDtypeStruct((3128, 8), jnp.float32),
    )(pooled_pad, cls_W1, cls_b1, cls_W2, cls_b2)[:G]
    return (logits, gate)


# speedup vs baseline: 3.3986x; 3.3986x over previous
import functools

import jax
import jax.numpy as jnp
import numpy as np
from jax import lax
from jax.experimental import pallas as pl
from jax.experimental.pallas import tpu as pltpu
from jax.experimental.pallas import tpu_sc as plsc

F = 16
G = 3125
VOCAB = 1000
E_DIM = 64
H = 128
HEADS = 4
N = F * G
E = 93750

NPAD = 50176
CH = 6272
EPAD = 94208
EPT = EPAD // 16
GR = EPT // 128
PAD_DST = 1 << 20


def _mm_kernel(x_ref, w_ref, o_ref):
    o_ref[...] = jnp.dot(x_ref[...], w_ref[...], preferred_element_type=jnp.float32)


def _mm(x, w, block_rows=2000):
    m, k = x.shape
    _, n = w.shape
    return pl.pallas_call(
        _mm_kernel,
        grid=(m // block_rows,),
        in_specs=[
            pl.BlockSpec((block_rows, k), lambda i: (i, 0)),
            pl.BlockSpec((k, n), lambda i: (0, 0)),
        ],
        out_specs=pl.BlockSpec((block_rows, n), lambda i: (i, 0)),
        out_shape=jax.ShapeDtypeStruct((m, n), jnp.float32),
    )(x, w)


def _mmp_kernel(x_ref, w_ref, o_ref):
    o_ref[...] = jnp.dot(x_ref[...], w_ref[...],
                         preferred_element_type=jnp.float32)[None]


def _mm_planar(x_pad, w, heads):
    k = x_pad.shape[1]
    br = 6272
    return pl.pallas_call(
        _mmp_kernel,
        grid=(heads, NPAD // br),
        in_specs=[
            pl.BlockSpec((br, k), lambda h, i: (i, 0)),
            pl.BlockSpec((k, H), lambda h, i: (0, h)),
        ],
        out_specs=pl.BlockSpec((1, br, H), lambda h, i: (h, i, 0)),
        out_shape=jax.ShapeDtypeStruct((heads, NPAD, H), jnp.float32),
    )(x_pad, w)


def _cls_kernel(p_ref, w1_ref, b1_ref, w2_ref, b2_ref, o_ref):
    h1 = jnp.dot(p_ref[...], w1_ref[...], preferred_element_type=jnp.float32)
    h1 = h1 + b1_ref[...]
    h1 = jnp.where(h1 > 0, h1, 0.01 * h1)
    o_ref[...] = jnp.dot(h1, w2_ref[...], preferred_element_type=jnp.float32) + b2_ref[...]


def _make_edge_kernel(heads):
    HN = heads * NPAD
    GSZ = 128
    NG = EPT // GSZ
    NP2 = NG // 2

    mesh = plsc.VectorSubcoreMesh(core_axis_name="c", subcore_axis_name="s",
                                  num_cores=2, num_subcores=16)

    @functools.partial(
        pl.kernel,
        out_type=(jax.ShapeDtypeStruct((HN, H), jnp.float32),
                  jax.ShapeDtypeStruct((HN,), jnp.float32)),
        mesh=mesh,
        scratch_types=[
            pltpu.VMEM((EPT,), jnp.int32),
            pltpu.VMEM((EPT,), jnp.int32),
            pltpu.VMEM((EPT,), jnp.int32),
            pltpu.VMEM((EPT,), jnp.float32),
            pltpu.VMEM((2 * GSZ,), jnp.int32),
            pltpu.VMEM((2 * GSZ,), jnp.int32),
            pltpu.VMEM((2 * GSZ,), jnp.float32),
            pltpu.VMEM((2 * GSZ,), jnp.float32),
            pltpu.VMEM((GSZ,), jnp.int32),
            pltpu.VMEM((GSZ,), jnp.int32),
            pltpu.VMEM((GSZ,), jnp.float32),
            pltpu.VMEM((GSZ,), jnp.float32),
            pltpu.VMEM((GSZ, H), jnp.float32),
            pltpu.VMEM((GSZ, H), jnp.float32),
            pltpu.VMEM((32, H), jnp.float32),
            pltpu.VMEM((128,), jnp.float32),
            pltpu.VMEM_SHARED((CH + 8, H), jnp.float32),
            pltpu.VMEM_SHARED((CH + 8,), jnp.float32),
            pltpu.SemaphoreType.DMA,
            pltpu.SemaphoreType.DMA,
        ],
    )
    def edge_kernel(h_hbm, asrc_hbm, adst_hbm, src_hbm, dst_hbm,
                    out_hbm, den_hbm,
                    src_v, dst_v, gsrc_v, exa_v, abi0, abi1, abv0, abv1,
                    gdl0, gdl1, gex0, gex1, rows0, rows1,
                    zer_v, zvec_v, acc, dacc, sem, sem2):
        sc = lax.axis_index("c")
        s = lax.axis_index("s")
        zl = jnp.zeros((16,), jnp.float32)

        def zb(i, _):
            for j in range(H // 16):
                zer_v[i, pl.ds(j * 16, 16)] = zl
            return 0
        lax.fori_loop(0, 32, zb, 0)
        for j in range(8):
            zvec_v[pl.ds(j * 16, 16)] = zl

        pltpu.sync_copy(src_hbm.at[pl.ds(s * EPT, EPT)], src_v)
        pltpu.sync_copy(dst_hbm.at[pl.ds(s * EPT, EPT)], dst_v)

        def head_body(h, _):
            hNP = h * NPAD

            def pa_idx(g, abi):
                base = g * GSZ
                for j in range(GSZ // 16):
                    sv = src_v[pl.ds(base + j * 16, 16)]
                    dv = dst_v[pl.ds(base + j * 16, 16)]
                    abi[pl.ds(j * 16, 16)] = sv + hNP
                    abi[pl.ds(GSZ + j * 16, 16)] = (
                        jnp.minimum(dv, N - 1) + hNP)
                    gsrc_v[pl.ds(base + j * 16, 16)] = sv + hNP

            def pa_fire(abi, abv):
                pltpu.async_copy(asrc_hbm.at[abi.at[pl.ds(0, GSZ)]],
                                 abv.at[pl.ds(0, GSZ)], sem)
                pltpu.async_copy(adst_hbm.at[abi.at[pl.ds(GSZ, GSZ)]],
                                 abv.at[pl.ds(GSZ, GSZ)], sem)

            def pa_wait(abi, abv):
                pltpu.make_async_copy(asrc_hbm.at[abi.at[pl.ds(0, GSZ)]],
                                      abv.at[pl.ds(0, GSZ)], sem).wait()
                pltpu.make_async_copy(adst_hbm.at[abi.at[pl.ds(GSZ, GSZ)]],
                                      abv.at[pl.ds(GSZ, GSZ)], sem).wait()

            def pa_compute(g, abv):
                base = g * GSZ
                for j in range(GSZ // 16):
                    al = (abv[pl.ds(j * 16, 16)]
                          + abv[pl.ds(GSZ + j * 16, 16)])
                    al = jnp.where(al > 0, al, 0.2 * al)
                    exa_v[pl.ds(base + j * 16, 16)] = jnp.exp(al)

            pa_idx(0, abi0)
            pa_fire(abi0, abv0)

            def pa_pair(p, _):
                g0 = 2 * p
                pa_wait(abi0, abv0)
                pa_idx(g0 + 1, abi1)
                pa_fire(abi1, abv1)
                pa_compute(g0, abv0)
                pa_wait(abi1, abv1)

                @pl.when(g0 + 2 < NG)
                def _():
                    pa_idx(g0 + 2, abi0)
                    pa_fire(abi0, abv0)
                pa_compute(g0 + 1, abv1)
                return 0
            lax.fori_loop(0, NP2, pa_pair, 0)


            def chunk_body(cc, _):
                c = cc * 2 + sc
                lo = c * CH

                for k in range(12):
                    pltpu.sync_copy(zer_v, acc.at[pl.ds(s * 392 + k * 32, 32)])
                pltpu.sync_copy(zer_v.at[pl.ds(0, 8)],
                                acc.at[pl.ds(s * 392 + 384, 8)])
                for k in range(3):
                    pltpu.sync_copy(zvec_v,
                                    dacc.at[pl.ds(s * 392 + k * 128, 128)])
                pltpu.sync_copy(zvec_v.at[pl.ds(0, 8)],
                                dacc.at[pl.ds(s * 392 + 384, 8)])
                plsc.subcore_barrier()

                def rows_fire(g, rows):
                    pltpu.async_copy(
                        h_hbm.at[gsrc_v.at[pl.ds(g * GSZ, GSZ)]], rows, sem2)

                def rows_wait(g, rows):
                    pltpu.make_async_copy(
                        h_hbm.at[gsrc_v.at[pl.ds(g * GSZ, GSZ)]],
                        rows, sem2).wait()

                def process(g, rows, gdl, gex):
                    base = g * GSZ
                    for j in range(GSZ // 16):
                        dv = dst_v[pl.ds(base + j * 16, 16)]
                        m = (dv >= lo) & (dv < lo + CH)
                        gdl[pl.ds(j * 16, 16)] = jnp.where(m, dv - lo, CH)
                        gex[pl.ds(j * 16, 16)] = jnp.where(
                            m, exa_v[pl.ds(base + j * 16, 16)], 0.0)

                    def mb(t, _):
                        exv = gex[pl.ds(t * 16, 16)]
                        for l in range(16):
                            exs = jnp.full((16,), exv[l], jnp.float32)
                            for j2 in range(8):
                                r = rows[t * 16 + l, pl.ds(j2 * 16, 16)]
                                rows[t * 16 + l, pl.ds(j2 * 16, 16)] = r * exs
                        return 0
                    lax.fori_loop(0, GSZ // 16, mb, 0)
                    pltpu.sync_copy(rows, acc.at[gdl], add=True)
                    pltpu.sync_copy(gex, dacc.at[gdl], add=True)

                rows_fire(0, rows0)

                def pb_pair(p, _):
                    g0 = 2 * p
                    rows_wait(g0, rows0)
                    rows_fire(g0 + 1, rows1)
                    process(g0, rows0, gdl0, gex0)
                    rows_wait(g0 + 1, rows1)

                    @pl.when(g0 + 2 < NG)
                    def _():
                        rows_fire(g0 + 2, rows0)
                    process(g0 + 1, rows1, gdl1, gex1)
                    return 0
                lax.fori_loop(0, NP2, pb_pair, 0)
                plsc.subcore_barrier()

                rowbase = hNP + lo + s * 392
                for k in range(3):
                    pltpu.sync_copy(acc.at[pl.ds(s * 392 + k * 128, 128)],
                                    out_hbm.at[pl.ds(rowbase + k * 128, 128)])
                pltpu.sync_copy(acc.at[pl.ds(s * 392 + 384, 8)],
                                out_hbm.at[pl.ds(rowbase + 384, 8)])

                @pl.when(s == 0)
                def _():
                    pltpu.sync_copy(dacc.at[pl.ds(0, CH)],
                                    den_hbm.at[pl.ds(hNP + lo, CH)])
                plsc.subcore_barrier()
                return 0
            lax.fori_loop(0, 4, chunk_body, 0)
            return 0

        lax.fori_loop(0, heads, head_body, 0)

    return edge_kernel


def _gat_layer(x, src_p, dst_p, W, att_src, att_dst, heads):
    x_pad = jnp.pad(x, ((0, NPAD - N), (0, 0)))
    hp = _mm_planar(x_pad, W, heads)
    Wr = W.reshape(W.shape[0], heads, H)
    ws = jnp.einsum("khc,hc->kh", Wr, att_src)
    wd = jnp.einsum("khc,hc->kh", Wr, att_dst)
    a_src = x @ ws
    a_dst = x @ wd
    asrc_p = jnp.pad(a_src.T, ((0, 0), (0, NPAD - N))).reshape(-1)
    adst_p = jnp.pad(a_dst.T, ((0, 0), (0, NPAD - N))).reshape(-1)

    ek = _make_edge_kernel(heads)
    out_sc, den_sc = ek(hp.reshape(heads * NPAD, H), asrc_p, adst_p,
                        src_p, dst_p)
    out_sc = out_sc.reshape(heads, NPAD, H)[:, :N]
    den_sc = den_sc.reshape(heads, NPAD)[:, :N]

    exl = jnp.exp(jax.nn.leaky_relu(a_src + a_dst, 0.2))
    exl_t = exl.T
    num = out_sc + hp[:, :N] * exl_t[:, :, None]
    den = den_sc + exl_t + 1e-16
    return num / den[:, :, None]


def kernel(field_indices, edge_index, batch_idx, emb_tables, aligner_W, aligner_b, mask_logits, gat1_W, gat1_att_src, gat1_att_dst, gat1_b, gat2_W, gat2_att_src, gat2_att_dst, gat2_b, cls_W1, cls_b1, cls_W2, cls_b2):
    flat_tab = emb_tables.reshape(F * VOCAB, E_DIM)
    idx = field_indices.T + (jnp.arange(F) * VOCAB)[None, :]
    emb = jnp.take(flat_tab, idx.reshape(-1), axis=0)
    gate = jax.nn.sigmoid(mask_logits)
    x = _mm(emb, aligner_W) + aligner_b
    x = x * jnp.tile(gate, G)[:, None]

    src_p = jnp.concatenate(
        [edge_index[0].astype(jnp.int32), jnp.zeros((EPAD - E,), jnp.int32)])
    dst_p = jnp.concatenate(
        [edge_index[1].astype(jnp.int32),
         jnp.full((EPAD - E,), PAD_DST, jnp.int32)])

    o1 = _gat_layer(x, src_p, dst_p, gat1_W, gat1_att_src, gat1_att_dst, HEADS)
    x1 = jnp.concatenate([o1[h] for h in range(HEADS)], axis=1) + gat1_b
    x1 = jnp.where(x1 > 0, x1, jnp.expm1(x1))

    o2 = _gat_layer(x1, src_p, dst_p, gat2_W, gat2_att_src, gat2_att_dst, 1)
    x2 = o2[0] + gat2_b

    counts = jax.ops.segment_sum(jnp.ones((N,), jnp.float32), batch_idx, num_segments=G)
    pooled = jax.ops.segment_sum(x2, batch_idx, num_segments=G)
    pooled = pooled / jnp.clip(counts, 1.0, None)[:, None]

    pooled_pad = jnp.pad(pooled, ((0, 3), (0, 0)))
    logits = pl.pallas_call(
        _cls_kernel,
        in_specs=[
            pl.BlockSpec((3128, H), lambda: (0, 0)),
            pl.BlockSpec((H, H // 2), lambda: (0, 0)),
            pl.BlockSpec((H // 2,), lambda: (0,)),
            pl.BlockSpec((H // 2, 8), lambda: (0, 0)),
            pl.BlockSpec((8,), lambda: (0,)),
        ],
        out_specs=pl.BlockSpec((3128, 8), lambda: (0, 0)),
        out_shape=jax.ShapeDtypeStruct((3128, 8), jnp.float32),
    )(pooled_pad, cls_W1, cls_b1, cls_W2, cls_b2)[:G]
    return (logits, gate)


# speedup vs baseline: 3.4698x; 1.0210x over previous
import functools

import jax
import jax.numpy as jnp
import numpy as np
from jax import lax
from jax.experimental import pallas as pl
from jax.experimental.pallas import tpu as pltpu
from jax.experimental.pallas import tpu_sc as plsc

F = 16
G = 3125
VOCAB = 1000
E_DIM = 64
H = 128
HEADS = 4
N = F * G
E = 93750

NPAD = 50176
CH = 6272
EPAD = 94208
EPT = EPAD // 16
GR = EPT // 128
PAD_DST = 1 << 20


def _mm_kernel(x_ref, w_ref, o_ref):
    o_ref[...] = jnp.dot(x_ref[...], w_ref[...], preferred_element_type=jnp.float32)


def _mm(x, w, block_rows=2000):
    m, k = x.shape
    _, n = w.shape
    return pl.pallas_call(
        _mm_kernel,
        grid=(m // block_rows,),
        in_specs=[
            pl.BlockSpec((block_rows, k), lambda i: (i, 0)),
            pl.BlockSpec((k, n), lambda i: (0, 0)),
        ],
        out_specs=pl.BlockSpec((block_rows, n), lambda i: (i, 0)),
        out_shape=jax.ShapeDtypeStruct((m, n), jnp.float32),
    )(x, w)


def _mmp_kernel(x_ref, w_ref, o_ref):
    o_ref[...] = jnp.dot(x_ref[...], w_ref[...],
                         preferred_element_type=jnp.float32)[None]


def _mm_planar(x_pad, w, heads):
    k = x_pad.shape[1]
    br = 6272
    return pl.pallas_call(
        _mmp_kernel,
        grid=(heads, NPAD // br),
        in_specs=[
            pl.BlockSpec((br, k), lambda h, i: (i, 0)),
            pl.BlockSpec((k, H), lambda h, i: (0, h)),
        ],
        out_specs=pl.BlockSpec((1, br, H), lambda h, i: (h, i, 0)),
        out_shape=jax.ShapeDtypeStruct((heads, NPAD, H), jnp.float32),
    )(x_pad, w)


def _cls_kernel(p_ref, w1_ref, b1_ref, w2_ref, b2_ref, o_ref):
    h1 = jnp.dot(p_ref[...], w1_ref[...], preferred_element_type=jnp.float32)
    h1 = h1 + b1_ref[...]
    h1 = jnp.where(h1 > 0, h1, 0.01 * h1)
    o_ref[...] = jnp.dot(h1, w2_ref[...], preferred_element_type=jnp.float32) + b2_ref[...]


def _make_edge_kernel(heads):
    HN = heads * NPAD
    GSZ = 128
    NG = EPT // GSZ
    NP2 = NG // 2

    mesh = plsc.VectorSubcoreMesh(core_axis_name="c", subcore_axis_name="s",
                                  num_cores=2, num_subcores=16)

    @functools.partial(
        pl.kernel,
        out_type=(jax.ShapeDtypeStruct((HN, H), jnp.float32),
                  jax.ShapeDtypeStruct((HN,), jnp.float32)),
        mesh=mesh,
        scratch_types=[
            pltpu.VMEM((EPT,), jnp.int32),
            pltpu.VMEM((EPT,), jnp.int32),
            pltpu.VMEM((EPT,), jnp.int32),
            pltpu.VMEM((EPT,), jnp.float32),
            pltpu.VMEM((2 * GSZ,), jnp.int32),
            pltpu.VMEM((2 * GSZ,), jnp.int32),
            pltpu.VMEM((2 * GSZ,), jnp.float32),
            pltpu.VMEM((2 * GSZ,), jnp.float32),
            pltpu.VMEM((GSZ,), jnp.int32),
            pltpu.VMEM((GSZ,), jnp.int32),
            pltpu.VMEM((GSZ,), jnp.int32),
            pltpu.VMEM((GSZ,), jnp.float32),
            pltpu.VMEM((GSZ,), jnp.float32),
            pltpu.VMEM((GSZ,), jnp.float32),
            pltpu.VMEM((GSZ, H), jnp.float32),
            pltpu.VMEM((GSZ, H), jnp.float32),
            pltpu.VMEM((GSZ, H), jnp.float32),
            pltpu.VMEM((16, H), jnp.float32),
            pltpu.VMEM((128,), jnp.float32),
            pltpu.VMEM_SHARED((CH + 8, H), jnp.float32),
            pltpu.VMEM_SHARED((CH + 8,), jnp.float32),
            pltpu.SemaphoreType.DMA,
            pltpu.SemaphoreType.DMA,
            pltpu.SemaphoreType.DMA,
            pltpu.SemaphoreType.DMA,
            pltpu.SemaphoreType.DMA,
            pltpu.SemaphoreType.DMA,
            pltpu.SemaphoreType.DMA,
        ],
    )
    def edge_kernel(h_hbm, asrc_hbm, adst_hbm, src_hbm, dst_hbm,
                    out_hbm, den_hbm,
                    src_v, dst_v, gsrc_v, exa_v, abi0, abi1, abv0, abv1,
                    gdl0, gdl1, gdl2, gex0, gex1, gex2, rows0, rows1, rows2,
                    zer_v, zvec_v, acc, dacc, sem,
                    sg0, sg1, sg2, ss0, ss1, ss2):
        sc = lax.axis_index("c")
        s = lax.axis_index("s")
        zl = jnp.zeros((16,), jnp.float32)

        def zb(i, _):
            for j in range(H // 16):
                zer_v[i, pl.ds(j * 16, 16)] = zl
            return 0
        lax.fori_loop(0, 16, zb, 0)
        for j in range(8):
            zvec_v[pl.ds(j * 16, 16)] = zl

        pltpu.sync_copy(src_hbm.at[pl.ds(s * EPT, EPT)], src_v)
        pltpu.sync_copy(dst_hbm.at[pl.ds(s * EPT, EPT)], dst_v)

        def head_body(h, _):
            hNP = h * NPAD

            def pa_idx(g, abi):
                base = g * GSZ
                for j in range(GSZ // 16):
                    sv = src_v[pl.ds(base + j * 16, 16)]
                    dv = dst_v[pl.ds(base + j * 16, 16)]
                    abi[pl.ds(j * 16, 16)] = sv + hNP
                    abi[pl.ds(GSZ + j * 16, 16)] = (
                        jnp.minimum(dv, N - 1) + hNP)
                    gsrc_v[pl.ds(base + j * 16, 16)] = sv + hNP

            def pa_fire(abi, abv):
                pltpu.async_copy(asrc_hbm.at[abi.at[pl.ds(0, GSZ)]],
                                 abv.at[pl.ds(0, GSZ)], sem)
                pltpu.async_copy(adst_hbm.at[abi.at[pl.ds(GSZ, GSZ)]],
                                 abv.at[pl.ds(GSZ, GSZ)], sem)

            def pa_wait(abi, abv):
                pltpu.make_async_copy(asrc_hbm.at[abi.at[pl.ds(0, GSZ)]],
                                      abv.at[pl.ds(0, GSZ)], sem).wait()
                pltpu.make_async_copy(adst_hbm.at[abi.at[pl.ds(GSZ, GSZ)]],
                                      abv.at[pl.ds(GSZ, GSZ)], sem).wait()

            def pa_compute(g, abv):
                base = g * GSZ
                for j in range(GSZ // 16):
                    al = (abv[pl.ds(j * 16, 16)]
                          + abv[pl.ds(GSZ + j * 16, 16)])
                    al = jnp.where(al > 0, al, 0.2 * al)
                    exa_v[pl.ds(base + j * 16, 16)] = jnp.exp(al)

            pa_idx(0, abi0)
            pa_fire(abi0, abv0)

            def pa_pair(p, _):
                g0 = 2 * p
                pa_wait(abi0, abv0)
                pa_idx(g0 + 1, abi1)
                pa_fire(abi1, abv1)
                pa_compute(g0, abv0)
                pa_wait(abi1, abv1)

                @pl.when(g0 + 2 < NG)
                def _():
                    pa_idx(g0 + 2, abi0)
                    pa_fire(abi0, abv0)
                pa_compute(g0 + 1, abv1)
                return 0
            lax.fori_loop(0, NP2, pa_pair, 0)


            bufs = ((rows0, gdl0, gex0, sg0, ss0),
                    (rows1, gdl1, gex1, sg1, ss1),
                    (rows2, gdl2, gex2, sg2, ss2))

            def chunk_body(cc, _):
                c = cc * 2 + sc
                lo = c * CH

                for k in range(24):
                    pltpu.sync_copy(zer_v, acc.at[pl.ds(s * 392 + k * 16, 16)])
                pltpu.sync_copy(zer_v.at[pl.ds(0, 8)],
                                acc.at[pl.ds(s * 392 + 384, 8)])
                for k in range(3):
                    pltpu.sync_copy(zvec_v,
                                    dacc.at[pl.ds(s * 392 + k * 128, 128)])
                pltpu.sync_copy(zvec_v.at[pl.ds(0, 8)],
                                dacc.at[pl.ds(s * 392 + 384, 8)])
                plsc.subcore_barrier()

                def g_fire(g, bi):
                    rows, _, _, sg, _ = bufs[bi]
                    pltpu.async_copy(
                        h_hbm.at[gsrc_v.at[pl.ds(g * GSZ, GSZ)]], rows, sg)

                def g_wait(g, bi):
                    rows, _, _, sg, _ = bufs[bi]
                    pltpu.make_async_copy(
                        h_hbm.at[gsrc_v.at[pl.ds(g * GSZ, GSZ)]],
                        rows, sg).wait()

                def s_fire(bi):
                    rows, gdl, gex, _, ss = bufs[bi]
                    pltpu.sync_copy(rows, acc.at[gdl], add=True)
                    pltpu.sync_copy(gex, dacc.at[gdl], add=True)

                def s_wait(bi):
                    pass

                def compute(g, bi):
                    rows, gdl, gex, _, _ = bufs[bi]
                    base = g * GSZ
                    for j in range(GSZ // 16):
                        dv = dst_v[pl.ds(base + j * 16, 16)]
                        m = (dv >= lo) & (dv < lo + CH)
                        gdl[pl.ds(j * 16, 16)] = jnp.where(m, dv - lo, CH)
                        gex[pl.ds(j * 16, 16)] = jnp.where(
                            m, exa_v[pl.ds(base + j * 16, 16)], 0.0)

                    def mb(t, _):
                        exv = gex[pl.ds(t * 16, 16)]
                        for l in range(16):
                            exs = jnp.full((16,), exv[l], jnp.float32)
                            for j2 in range(8):
                                r = rows[t * 16 + l, pl.ds(j2 * 16, 16)]
                                rows[t * 16 + l, pl.ds(j2 * 16, 16)] = r * exs
                        return 0
                    lax.fori_loop(0, GSZ // 16, mb, 0)

                g_fire(0, 0)
                g_fire(1, 1)

                def first_body():
                    g_wait(0, 0)
                    g_fire(2, 2)
                    compute(0, 0)
                    s_fire(0)
                first_body()

                def triple(p, _):
                    g0 = 3 * p + 1

                    def tb(off):
                        g = g0 + off
                        bi = (1 + off) % 3

                        @pl.when(g < NG)
                        def _():
                            g_wait(g, bi)
                            nbi = (bi + 2) % 3
                            s_wait(nbi)

                            @pl.when(g + 2 < NG)
                            def _():
                                g_fire(g + 2, nbi)
                            compute(g, bi)
                            s_fire(bi)
                    tb(0)
                    tb(1)
                    tb(2)
                    return 0
                lax.fori_loop(0, (NG + 1) // 3, triple, 0)
                s_wait((NG - 1) % 3)
                plsc.subcore_barrier()

                rowbase = hNP + lo + s * 392
                for k in range(3):
                    pltpu.sync_copy(acc.at[pl.ds(s * 392 + k * 128, 128)],
                                    out_hbm.at[pl.ds(rowbase + k * 128, 128)])
                pltpu.sync_copy(acc.at[pl.ds(s * 392 + 384, 8)],
                                out_hbm.at[pl.ds(rowbase + 384, 8)])

                @pl.when(s == 0)
                def _():
                    pltpu.sync_copy(dacc.at[pl.ds(0, CH)],
                                    den_hbm.at[pl.ds(hNP + lo, CH)])
                plsc.subcore_barrier()
                return 0
            lax.fori_loop(0, 4, chunk_body, 0)
            return 0

        lax.fori_loop(0, heads, head_body, 0)

    return edge_kernel


def _gat_layer(x, src_p, dst_p, W, att_src, att_dst, heads):
    x_pad = jnp.pad(x, ((0, NPAD - N), (0, 0)))
    hp = _mm_planar(x_pad, W, heads)
    Wr = W.reshape(W.shape[0], heads, H)
    ws = jnp.einsum("khc,hc->kh", Wr, att_src)
    wd = jnp.einsum("khc,hc->kh", Wr, att_dst)
    a_src = x @ ws
    a_dst = x @ wd
    asrc_p = jnp.pad(a_src.T, ((0, 0), (0, NPAD - N))).reshape(-1)
    adst_p = jnp.pad(a_dst.T, ((0, 0), (0, NPAD - N))).reshape(-1)

    ek = _make_edge_kernel(heads)
    out_sc, den_sc = ek(hp.reshape(heads * NPAD, H), asrc_p, adst_p,
                        src_p, dst_p)
    out_sc = out_sc.reshape(heads, NPAD, H)[:, :N]
    den_sc = den_sc.reshape(heads, NPAD)[:, :N]

    exl = jnp.exp(jax.nn.leaky_relu(a_src + a_dst, 0.2))
    exl_t = exl.T
    num = out_sc + hp[:, :N] * exl_t[:, :, None]
    den = den_sc + exl_t + 1e-16
    return num / den[:, :, None]


def kernel(field_indices, edge_index, batch_idx, emb_tables, aligner_W, aligner_b, mask_logits, gat1_W, gat1_att_src, gat1_att_dst, gat1_b, gat2_W, gat2_att_src, gat2_att_dst, gat2_b, cls_W1, cls_b1, cls_W2, cls_b2):
    flat_tab = emb_tables.reshape(F * VOCAB, E_DIM)
    idx = field_indices.T + (jnp.arange(F) * VOCAB)[None, :]
    emb = jnp.take(flat_tab, idx.reshape(-1), axis=0)
    gate = jax.nn.sigmoid(mask_logits)
    x = _mm(emb, aligner_W) + aligner_b
    x = x * jnp.tile(gate, G)[:, None]

    src_p = jnp.concatenate(
        [edge_index[0].astype(jnp.int32), jnp.zeros((EPAD - E,), jnp.int32)])
    dst_p = jnp.concatenate(
        [edge_index[1].astype(jnp.int32),
         jnp.full((EPAD - E,), PAD_DST, jnp.int32)])

    o1 = _gat_layer(x, src_p, dst_p, gat1_W, gat1_att_src, gat1_att_dst, HEADS)
    x1 = jnp.concatenate([o1[h] for h in range(HEADS)], axis=1) + gat1_b
    x1 = jnp.where(x1 > 0, x1, jnp.expm1(x1))

    o2 = _gat_layer(x1, src_p, dst_p, gat2_W, gat2_att_src, gat2_att_dst, 1)
    x2 = o2[0] + gat2_b

    counts = jax.ops.segment_sum(jnp.ones((N,), jnp.float32), batch_idx, num_segments=G)
    pooled = jax.ops.segment_sum(x2, batch_idx, num_segments=G)
    pooled = pooled / jnp.clip(counts, 1.0, None)[:, None]

    pooled_pad = jnp.pad(pooled, ((0, 3), (0, 0)))
    logits = pl.pallas_call(
        _cls_kernel,
        in_specs=[
            pl.BlockSpec((3128, H), lambda: (0, 0)),
            pl.BlockSpec((H, H // 2), lambda: (0, 0)),
            pl.BlockSpec((H // 2,), lambda: (0,)),
            pl.BlockSpec((H // 2, 8), lambda: (0, 0)),
            pl.BlockSpec((8,), lambda: (0,)),
        ],
        out_specs=pl.BlockSpec((3128, 8), lambda: (0, 0)),
        out_shape=jax.ShapeDtypeStruct((3128, 8), jnp.float32),
    )(pooled_pad, cls_W1, cls_b1, cls_W2, cls_b2)[:G]
    return (logits, gate)
